# baseline (speedup 1.0000x reference)
import functools

import jax
import jax.numpy as jnp
from jax import lax
from jax.experimental import pallas as pl
from jax.experimental.pallas import tpu as pltpu
from jax.experimental.pallas import tpu_sc as plsc

EMBED_DIM = 32
BATCH = 16384

NC = 2
NS = 16
NW = NC * NS
BPW = BATCH // NW
CHUNK = 128
NCHUNK = BPW // CHUNK


def _sc_kernel(users_hbm, books_hbm, ut_hbm, bt_hbm, wb_hbm, out_hbm,
               idxu_v, idxb_v, rows_u, rows_b, wb_v, out_v, sem):
    wid = lax.axis_index("s") * NC + lax.axis_index("c")
    base = wid * BPW

    pltpu.sync_copy(users_hbm.at[wid], idxu_v)
    pltpu.sync_copy(books_hbm.at[wid], idxb_v)
    pltpu.sync_copy(wb_hbm, wb_v)

    copies = []
    for j in range(NCHUNK):
        copies.append(pltpu.async_copy(
            ut_hbm.at[idxu_v.at[j]], rows_u.at[pl.ds(j * CHUNK, CHUNK)], sem))
    for j in range(NCHUNK):
        copies.append(pltpu.async_copy(
            bt_hbm.at[idxb_v.at[j]], rows_b.at[pl.ds(j * CHUNK, CHUNK)], sem))
    for c in copies:
        c.wait()

    w_vecs = [wb_v[pl.ds(k * 16, 16)] for k in range(5)]
    w_all = [w_vecs[d // 16][d % 16] for d in range(2 * EMBED_DIM)]
    w_u = w_all[:EMBED_DIM]
    w_b = w_all[EMBED_DIM:]
    bias = w_vecs[4][0]

    lane = lax.iota(jnp.int32, 16)
    dvecs = [jnp.full((16,), d, jnp.int32) for d in range(EMBED_DIM)]

    def group(g, carry):
        rows = g * 16 + lane
        acc = bias + jnp.zeros((16,), jnp.float32)
        for d in range(EMBED_DIM):
            colu = plsc.load_gather(rows_u, [rows, dvecs[d]])
            acc = acc + colu * w_u[d]
            colb = plsc.load_gather(rows_b, [rows, dvecs[d]])
            acc = acc + colb * w_b[d]
        out_v[pl.ds(g * 16, 16)] = 5.0 / (1.0 + jnp.exp(-acc))
        return carry

    lax.fori_loop(0, BPW // 16, group, 0)

    pltpu.sync_copy(out_v, out_hbm.at[pl.ds(base, BPW)])


@jax.jit
def _run(users_r, books_r, user_table, book_table, wb):
    mesh = plsc.VectorSubcoreMesh(core_axis_name="c", subcore_axis_name="s")
    f = pl.kernel(
        _sc_kernel, mesh=mesh,
        out_type=jax.ShapeDtypeStruct((BATCH,), jnp.float32),
        scratch_types=[
            pltpu.VMEM((NCHUNK, CHUNK), jnp.int32),
            pltpu.VMEM((NCHUNK, CHUNK), jnp.int32),
            pltpu.VMEM((BPW, EMBED_DIM), jnp.float32),
            pltpu.VMEM((BPW, EMBED_DIM), jnp.float32),
            pltpu.VMEM((80,), jnp.float32),
            pltpu.VMEM((BPW,), jnp.float32),
            pltpu.SemaphoreType.DMA,
        ],
        compiler_params=pltpu.CompilerParams(
            use_tc_tiling_on_sc=False,
            needs_layout_passes=False,
        ),
    )
    return f(users_r, books_r, user_table, book_table, wb)


def kernel(users, books, user_table, book_table, W, b):
    users_r = users.astype(jnp.int32).reshape(NW, NCHUNK, CHUNK)
    books_r = books.astype(jnp.int32).reshape(NW, NCHUNK, CHUNK)
    wb = jnp.zeros((80,), jnp.float32).at[:65].set(
        jnp.concatenate([W.reshape(-1), b]).astype(jnp.float32))
    out = _run(users_r, books_r, user_table, book_table, wb)
    return out.reshape(BATCH, 1)


# speedup vs baseline: 5.8315x; 5.8315x over previous
import jax
import jax.numpy as jnp
from jax import lax
from jax.experimental import pallas as pl
from jax.experimental.pallas import tpu as pltpu
from jax.experimental.pallas import tpu_sc as plsc

EMBED_DIM = 32
BATCH = 16384
NROWS = 1000000

NC = 2
NS = 16
NW = NC * NS
BPW = BATCH // NW
CHUNK = 128
NCHUNK = BPW // CHUNK

BLK = 8192
GRID = (NROWS + BLK - 1) // BLK


def _tc_body(w_ref, u_ref, b_ref, su_ref, sb_ref):
    w = w_ref[...]
    wu = w[:EMBED_DIM]
    wb = w[EMBED_DIM:2 * EMBED_DIM]
    bias = w[2 * EMBED_DIM, 0]
    su_ref[...] = jnp.sum(u_ref[...] * wu, axis=0) + bias
    sb_ref[...] = jnp.sum(b_ref[...] * wb, axis=0)


def _sc_body(users_hbm, books_hbm, su_hbm, sb_hbm, out_hbm,
             uidx_v, bidx_v, vals_u, vals_b, out_v, sem):
    wid = lax.axis_index("s") * NC + lax.axis_index("c")
    base = wid * BPW

    pltpu.sync_copy(users_hbm.at[wid], uidx_v)
    pltpu.sync_copy(books_hbm.at[wid], bidx_v)

    copies = []
    for j in range(NCHUNK):
        lo = j * CHUNK
        copies.append(pltpu.async_copy(
            su_hbm.at[uidx_v.at[pl.ds(lo, CHUNK)]], vals_u.at[pl.ds(lo, CHUNK)], sem))
        copies.append(pltpu.async_copy(
            sb_hbm.at[bidx_v.at[pl.ds(lo, CHUNK)]], vals_b.at[pl.ds(lo, CHUNK)], sem))
    for c in copies:
        c.wait()

    def group(g, carry):
        vu = vals_u[pl.ds(g * 16, 16)]
        vb = vals_b[pl.ds(g * 16, 16)]
        out_v[pl.ds(g * 16, 16)] = 5.0 / (1.0 + jnp.exp(-(vu + vb)))
        return carry

    lax.fori_loop(0, BPW // 16, group, 0)

    pltpu.sync_copy(out_v, out_hbm.at[pl.ds(base, BPW)])


@jax.jit
def _run(users_r, books_r, ut_t, bt_t, wcol):
    su, sb = pl.pallas_call(
        _tc_body,
        grid=(GRID,),
        in_specs=[
            pl.BlockSpec((128, 1), lambda j: (0, 0)),
            pl.BlockSpec((EMBED_DIM, BLK), lambda j: (0, j)),
            pl.BlockSpec((EMBED_DIM, BLK), lambda j: (0, j)),
        ],
        out_specs=[
            pl.BlockSpec((BLK,), lambda j: (j,)),
            pl.BlockSpec((BLK,), lambda j: (j,)),
        ],
        out_shape=[
            jax.ShapeDtypeStruct((NROWS,), jnp.float32),
            jax.ShapeDtypeStruct((NROWS,), jnp.float32),
        ],
    )(wcol, ut_t, bt_t)

    mesh = plsc.VectorSubcoreMesh(core_axis_name="c", subcore_axis_name="s")
    out = pl.kernel(
        _sc_body, mesh=mesh,
        out_type=jax.ShapeDtypeStruct((BATCH,), jnp.float32),
        scratch_types=[
            pltpu.VMEM((BPW,), jnp.int32),
            pltpu.VMEM((BPW,), jnp.int32),
            pltpu.VMEM((BPW,), jnp.float32),
            pltpu.VMEM((BPW,), jnp.float32),
            pltpu.VMEM((BPW,), jnp.float32),
            pltpu.SemaphoreType.DMA,
        ],
        compiler_params=pltpu.CompilerParams(
            needs_layout_passes=False,
        ),
    )(users_r, books_r, su, sb)
    return out


def kernel(users, books, user_table, book_table, W, b):
    users_r = users.astype(jnp.int32).reshape(NW, BPW)
    books_r = books.astype(jnp.int32).reshape(NW, BPW)
    wcol = jnp.zeros((128, 1), jnp.float32).at[:65, 0].set(
        jnp.concatenate([W.reshape(-1), b]).astype(jnp.float32))
    out = _run(users_r, books_r, user_table.T, book_table.T, wcol)
    return out.reshape(BATCH, 1)


# speedup vs baseline: 8.4723x; 1.4529x over previous
import jax
import jax.numpy as jnp
from jax import lax
from jax.experimental import pallas as pl
from jax.experimental.pallas import tpu as pltpu
from jax.experimental.pallas import tpu_sc as plsc

EMBED_DIM = 32
BATCH = 16384
NROWS = 1000000

NC = 2
NS = 16
NW = NC * NS
BPW = BATCH // NW
CHUNK = 128
NCHUNK = BPW // CHUNK

BLK = 32768
GRID = (NROWS + BLK - 1) // BLK


def _tc_body(w_ref, u_ref, b_ref, su_ref, sb_ref):
    w = w_ref[...]
    wu = w[:EMBED_DIM]
    wb = w[EMBED_DIM:2 * EMBED_DIM]
    bias = w[2 * EMBED_DIM, 0]
    su_ref[...] = jnp.sum(u_ref[...] * wu, axis=0) + bias
    sb_ref[...] = jnp.sum(b_ref[...] * wb, axis=0)


def _sc_body(users_hbm, books_hbm, su_hbm, sb_hbm, out_hbm,
             uidx_v, bidx_v, vals_u, vals_b, out_v, sem):
    wid = lax.axis_index("s") * NC + lax.axis_index("c")
    base = wid * BPW

    pltpu.sync_copy(users_hbm.at[wid], uidx_v)
    pltpu.sync_copy(books_hbm.at[wid], bidx_v)

    copies = []
    for j in range(NCHUNK):
        lo = j * CHUNK
        copies.append(pltpu.async_copy(
            su_hbm.at[uidx_v.at[pl.ds(lo, CHUNK)]], vals_u.at[pl.ds(lo, CHUNK)], sem))
        copies.append(pltpu.async_copy(
            sb_hbm.at[bidx_v.at[pl.ds(lo, CHUNK)]], vals_b.at[pl.ds(lo, CHUNK)], sem))
    for c in copies:
        c.wait()

    def group(g, carry):
        vu = vals_u[pl.ds(g * 16, 16)]
        vb = vals_b[pl.ds(g * 16, 16)]
        out_v[pl.ds(g * 16, 16)] = 5.0 / (1.0 + jnp.exp(-(vu + vb)))
        return carry

    lax.fori_loop(0, BPW // 16, group, 0)

    pltpu.sync_copy(out_v, out_hbm.at[pl.ds(base, BPW)])


@jax.jit
def _run(users_r, books_r, ut_t, bt_t, wcol):
    su, sb = pl.pallas_call(
        _tc_body,
        grid=(GRID,),
        in_specs=[
            pl.BlockSpec((128, 1), lambda j: (0, 0)),
            pl.BlockSpec((EMBED_DIM, BLK), lambda j: (0, j)),
            pl.BlockSpec((EMBED_DIM, BLK), lambda j: (0, j)),
        ],
        out_specs=[
            pl.BlockSpec((BLK,), lambda j: (j,)),
            pl.BlockSpec((BLK,), lambda j: (j,)),
        ],
        out_shape=[
            jax.ShapeDtypeStruct((NROWS,), jnp.float32),
            jax.ShapeDtypeStruct((NROWS,), jnp.float32),
        ],
    )(wcol, ut_t, bt_t)

    mesh = plsc.VectorSubcoreMesh(core_axis_name="c", subcore_axis_name="s")
    out = pl.kernel(
        _sc_body, mesh=mesh,
        out_type=jax.ShapeDtypeStruct((BATCH,), jnp.float32),
        scratch_types=[
            pltpu.VMEM((BPW,), jnp.int32),
            pltpu.VMEM((BPW,), jnp.int32),
            pltpu.VMEM((BPW,), jnp.float32),
            pltpu.VMEM((BPW,), jnp.float32),
            pltpu.VMEM((BPW,), jnp.float32),
            pltpu.SemaphoreType.DMA,
        ],
        compiler_params=pltpu.CompilerParams(
            needs_layout_passes=False,
        ),
    )(users_r, books_r, su, sb)
    return out


def kernel(users, books, user_table, book_table, W, b):
    users_r = users.astype(jnp.int32).reshape(NW, BPW)
    books_r = books.astype(jnp.int32).reshape(NW, BPW)
    wcol = jnp.zeros((128, 1), jnp.float32).at[:65, 0].set(
        jnp.concatenate([W.reshape(-1), b]).astype(jnp.float32))
    out = _run(users_r, books_r, user_table.T, book_table.T, wcol)
    return out.reshape(BATCH, 1)
